# baseline (speedup 1.0000x reference)
import jax
import jax.numpy as jnp
from jax.experimental import pallas as pl
from jax.experimental.pallas import tpu as pltpu

N, E = 2048, 4096
DV, DE, NHID, NCLUST = 128, 16, 64, 10
ALPHA = 0.2

BM = 256
BE = 256
NB = N // BM
EB = E // BE

_CPARAMS = pltpu.CompilerParams(
    dimension_semantics=("arbitrary",),
    vmem_limit_bytes=60 * 1024 * 1024,
)


def _node1_kernel(T_ref, He_ref, p_ref, adj_ref, Hv_ref, W_ref, b_ref,
                  out_ref, acc):
    i = pl.program_id(0)

    @pl.when(i < NB)
    def _():
        d = jax.lax.dot_general(p_ref[...], He_ref[...],
                                (((1,), (1,)), ((), ())),
                                preferred_element_type=jnp.float32)
        Tj = T_ref[pl.ds(i * BM, BM), :]
        mult = jax.lax.dot_general(T_ref[...], Tj * d,
                                   (((1,), (1,)), ((), ())),
                                   preferred_element_type=jnp.float32)
        rows = jax.lax.broadcasted_iota(jnp.int32, (N, BM), 0)
        cols = i * BM + jax.lax.broadcasted_iota(jnp.int32, (N, BM), 1)
        adjcb = jnp.where(rows == cols, adj_ref[...], mult * adj_ref[...])
        colmax = jnp.max(adjcb, axis=0, keepdims=True)
        Xb = jax.lax.dot_general(Hv_ref[pl.ds(i * BM, BM), :], W_ref[...],
                                 (((1,), (0,)), ((), ())),
                                 preferred_element_type=jnp.float32)
        contrib = jax.lax.dot_general(adjcb * (1.0 / colmax), Xb,
                                      (((1,), (0,)), ((), ())),
                                      preferred_element_type=jnp.float32)

        @pl.when(i == 0)
        def _():
            acc[...] = contrib

        @pl.when(i != 0)
        def _():
            acc[...] = acc[...] + contrib

    @pl.when(i == NB)
    def _():
        out_ref[...] = acc[...] + b_ref[...]


def _edge_kernel(T_ref, Hv_ref, p2_ref, eadj_ref, He_ref, W2_ref, b2_ref,
                 p3_ref, out_ref, acc):
    i = pl.program_id(0)

    @pl.when(i < EB)
    def _():
        d = jax.lax.dot_general(Hv_ref[...], p2_ref[...],
                                (((1,), (1,)), ((), ())),
                                preferred_element_type=jnp.float32)
        db = d.astype(jnp.bfloat16)
        Tj = T_ref[:, pl.ds(i * BE, BE)]
        mult = jax.lax.dot_general(T_ref[...], Tj * db,
                                   (((0,), (0,)), ((), ())),
                                   preferred_element_type=jnp.float32)
        rows = jax.lax.broadcasted_iota(jnp.int32, (E, BE), 0)
        cols = i * BE + jax.lax.broadcasted_iota(jnp.int32, (E, BE), 1)
        adjcb = jnp.where(rows == cols, eadj_ref[...], mult * eadj_ref[...])
        colmax = jnp.max(adjcb, axis=0, keepdims=True)
        w23 = jax.lax.dot_general(W2_ref[...], p3_ref[...],
                                  (((1,), (1,)), ((), ())),
                                  preferred_element_type=jnp.float32)
        yb = jax.lax.dot_general(He_ref[pl.ds(i * BE, BE), :], w23,
                                 (((1,), (0,)), ((), ())),
                                 preferred_element_type=jnp.float32)
        contrib = jax.lax.dot_general(adjcb * (1.0 / colmax), yb,
                                      (((1,), (0,)), ((), ())),
                                      preferred_element_type=jnp.float32)

        @pl.when(i == 0)
        def _():
            acc[...] = contrib

        @pl.when(i != 0)
        def _():
            acc[...] = acc[...] + contrib

    @pl.when(i == EB)
    def _():
        c23 = jax.lax.dot_general(b2_ref[...], p3_ref[...],
                                  (((1,), (1,)), ((), ())),
                                  preferred_element_type=jnp.float32)
        out_ref[...] = acc[...] + c23


def _node3_q_kernel(T_ref, d_ref, adj_ref, Hv_ref, W_ref, b_ref, mu_ref,
                    x_ref, q_ref, acc):
    i = pl.program_id(0)

    @pl.when(i < NB)
    def _():
        d = d_ref[...]
        Tj = T_ref[pl.ds(i * BM, BM), :]
        mult = jax.lax.dot_general(T_ref[...], Tj * d,
                                   (((1,), (1,)), ((), ())),
                                   preferred_element_type=jnp.float32)
        rows = jax.lax.broadcasted_iota(jnp.int32, (N, BM), 0)
        cols = i * BM + jax.lax.broadcasted_iota(jnp.int32, (N, BM), 1)
        adjcb = jnp.where(rows == cols, adj_ref[...], mult * adj_ref[...])
        colmax = jnp.max(adjcb, axis=0, keepdims=True)
        Xb = jax.lax.dot_general(Hv_ref[pl.ds(i * BM, BM), :], W_ref[...],
                                 (((1,), (0,)), ((), ())),
                                 preferred_element_type=jnp.float32)
        contrib = jax.lax.dot_general(adjcb * (1.0 / colmax), Xb,
                                      (((1,), (0,)), ((), ())),
                                      preferred_element_type=jnp.float32)

        @pl.when(i == 0)
        def _():
            acc[...] = contrib

        @pl.when(i != 0)
        def _():
            acc[...] = acc[...] + contrib

    @pl.when(i == NB)
    def _():
        x = acc[...] + b_ref[...]
        x_ref[...] = x
        mu = mu_ref[...]
        x2 = jnp.sum(x * x, axis=1, keepdims=True)
        mu2 = jnp.sum(mu * mu, axis=1)[None, :]
        cross = jax.lax.dot_general(x, mu, (((1,), (1,)), ((), ())),
                                    preferred_element_type=jnp.float32)
        dist = x2 - 2.0 * cross + mu2
        q = 1.0 / (1.0 + dist / ALPHA + 1e-8)
        q = q ** (ALPHA + 1.0) / 2.0
        q_ref[...] = q / jnp.sum(q, axis=1, keepdims=True)


def kernel(features, edge_features, adj, edge_adj, Tmat,
           W1, b1, p1, W2, b2, p2, W3, b3, p3, mu):
    f32 = jnp.float32
    Tbf = Tmat.astype(jnp.bfloat16)

    def full(shape):
        return pl.BlockSpec(shape, lambda i: (0,) * len(shape))

    Xh1 = pl.pallas_call(
        _node1_kernel,
        grid=(NB + 1,),
        in_specs=[full((N, E)), full((E, DE)), full((1, DE)),
                  pl.BlockSpec((N, BM), lambda i: (0, jnp.minimum(i, NB - 1))),
                  full((N, DV)), full((DV, NHID)), full((1, NHID))],
        out_specs=full((N, NHID)),
        out_shape=jax.ShapeDtypeStruct((N, NHID), f32),
        scratch_shapes=[pltpu.VMEM((N, NHID), f32)],
        compiler_params=_CPARAMS,
    )(Tmat, edge_features, p1, adj, features, W1, b1.reshape(1, NHID))

    d3 = pl.pallas_call(
        _edge_kernel,
        grid=(EB + 1,),
        in_specs=[full((N, E)), full((N, NHID)), full((1, NHID)),
                  pl.BlockSpec((E, BE), lambda i: (0, jnp.minimum(i, EB - 1))),
                  full((E, DE)), full((DE, DE)), full((1, DE)), full((1, DE))],
        out_specs=full((E, 1)),
        out_shape=jax.ShapeDtypeStruct((E, 1), f32),
        scratch_shapes=[pltpu.VMEM((E, 1), f32)],
        compiler_params=_CPARAMS,
    )(Tbf, Xh1, p2, edge_adj, edge_features, W2, b2.reshape(1, DE), p3)

    x, q = pl.pallas_call(
        _node3_q_kernel,
        grid=(NB + 1,),
        in_specs=[full((N, E)), full((1, E)),
                  pl.BlockSpec((N, BM), lambda i: (0, jnp.minimum(i, NB - 1))),
                  full((N, NHID)), full((NHID, DV)), full((1, DV)),
                  full((NCLUST, DV))],
        out_specs=[full((N, DV)), full((N, NCLUST))],
        out_shape=[jax.ShapeDtypeStruct((N, DV), f32),
                   jax.ShapeDtypeStruct((N, NCLUST), f32)],
        scratch_shapes=[pltpu.VMEM((N, DV), f32)],
        compiler_params=_CPARAMS,
    )(Tmat, d3.reshape(1, E), adj, Xh1, W3, b3.reshape(1, DV), mu)

    return (x, q)
